# baseline (speedup 1.0000x reference)
import jax
import jax.numpy as jnp
from jax.experimental import pallas as pl


def _add_block(x_ref, pos_ref, o_ref):
    o_ref[...] = x_ref[...] + pos_ref[...][None]


def kernel(x, pos_table):
    B, S, N = x.shape
    BS = 1024
    grid = (S // BS, B)
    return pl.pallas_call(
        _add_block,
        grid=grid,
        in_specs=[
            pl.BlockSpec((1, BS, N), lambda s, b: (b, s, 0)),
            pl.BlockSpec((BS, N), lambda s, b: (s, 0)),
        ],
        out_specs=pl.BlockSpec((1, BS, N), lambda s, b: (b, s, 0)),
        out_shape=jax.ShapeDtypeStruct((B, S, N), x.dtype),
    )(x, pos_table)
